# baseline (speedup 1.0000x reference)
import jax
import jax.numpy as jnp
from jax import lax
from jax.experimental import pallas as pl
from jax.experimental.pallas import tpu as pltpu
from jax.experimental.pallas import tpu_sc as plsc

G = 4096
IN = 64
OUT = 64
B = 16384
NC = 2
NS = 16
NW = NC * NS
BPW = B // NW
BLK = 128
NBLK = BPW // BLK
ROWS = G + 1


def _lookup_body(x_hbm, fpbf_hbm, borders_hbm, invlen_hbm, out_hbm,
                 borders_v, invlen_v, x_v, idxl_v, idxr_v, delta_v,
                 bufl_v, bufr_v, acc_v, acct_v, semx, seml, semr):
    wid = lax.axis_index("s") * NC + lax.axis_index("c")
    base = wid * BPW
    iota16 = lax.iota(jnp.int32, 16)

    pltpu.sync_copy(borders_hbm, borders_v)
    pltpu.sync_copy(invlen_hbm, invlen_v)

    zeros16 = jnp.zeros((16,), jnp.float32)

    @plsc.parallel_loop(0, BPW, unroll=4)
    def _(b):
        for r in range(OUT // 16):
            acc_v[b, pl.ds(r * 16, 16)] = zeros16

    def compute_weights(i1, par):
        def wgt_body(j, cc):
            xv = x_v[par, pl.ds(j * 16, 16)]
            e = jnp.exp(-jnp.abs(xv))
            cdf = jnp.where(xv > 0.0, 1.0 - 0.5 * e, 0.5 * e)
            idx = jnp.clip((cdf * float(G)).astype(jnp.int32), 0, G - 1)
            left = plsc.load_gather(borders_v, [idx])
            invl = plsc.load_gather(invlen_v, [idx])
            delta_v[par, pl.ds(j * 16, 16)] = (xv - left) * invl
            row = idx * IN + i1
            idxl_v[par, pl.ds(j * 16, 16)] = row
            idxr_v[par, pl.ds(j * 16, 16)] = row + IN
            return cc
        lax.fori_loop(0, BPW // 16, wgt_body, 0)

    def fire_block(par, blk):
        pltpu.async_copy(
            fpbf_hbm.at[idxl_v.at[par, pl.ds(blk * BLK, BLK)]],
            bufl_v.at[pl.ds(blk * BLK, BLK)], seml)
        pltpu.async_copy(
            fpbf_hbm.at[idxr_v.at[par, pl.ds(blk * BLK, BLK)]],
            bufr_v.at[pl.ds(blk * BLK, BLK)], semr)

    def wait_block(par, blk):
        pltpu.make_async_copy(
            fpbf_hbm.at[idxl_v.at[par, pl.ds(blk * BLK, BLK)]],
            bufl_v.at[pl.ds(blk * BLK, BLK)], seml).wait()
        pltpu.make_async_copy(
            fpbf_hbm.at[idxr_v.at[par, pl.ds(blk * BLK, BLK)]],
            bufr_v.at[pl.ds(blk * BLK, BLK)], semr).wait()

    pltpu.sync_copy(x_hbm.at[0, pl.ds(base, BPW)], x_v.at[0])
    pltpu.async_copy(x_hbm.at[1, pl.ds(base, BPW)], x_v.at[1], semx)
    compute_weights(0, 0)
    for blk in range(NBLK):
        fire_block(0, blk)

    def dim_body(i, c):
        par = lax.rem(i, 2)
        parn = 1 - par

        @pl.when(i < IN - 1)
        def _():
            pltpu.make_async_copy(
                x_hbm.at[i + 1, pl.ds(base, BPW)], x_v.at[parn], semx).wait()

            @pl.when(i < IN - 2)
            def _():
                pltpu.async_copy(
                    x_hbm.at[i + 2, pl.ds(base, BPW)], x_v.at[par], semx)

            compute_weights(i + 1, parn)

        for blk in range(NBLK):
            wait_block(par, blk)

            @plsc.parallel_loop(0, BLK, unroll=8)
            def _(j):
                b = blk * BLK + j
                d = plsc.load_gather(
                    delta_v.at[par], [jnp.full((16,), b, jnp.int32)])
                for grp in range(2):
                    lw = plsc.bitcast(bufl_v[b, pl.ds(grp * 32, 32)],
                                      jnp.int32)
                    rw = plsc.bitcast(bufr_v[b, pl.ds(grp * 32, 32)],
                                      jnp.int32)
                    for half in range(2):
                        if half == 0:
                            li = lax.shift_left(lw, 16)
                            ri = lax.shift_left(rw, 16)
                        else:
                            li = lax.bitwise_and(lw, jnp.int32(-65536))
                            ri = lax.bitwise_and(rw, jnp.int32(-65536))
                        L = plsc.bitcast(li, jnp.float32)
                        R = plsc.bitcast(ri, jnp.float32)
                        r = grp * 2 + half
                        plsc.addupdate(acc_v.at[b, pl.ds(r * 16, 16)],
                                       L + d * (R - L))

            @pl.when(i < IN - 1)
            def _():
                fire_block(parn, blk)

        return c

    lax.fori_loop(0, IN, dim_body, 0)

    @plsc.parallel_loop(0, OUT * (BPW // 16), unroll=4)
    def _(t):
        o = lax.shift_right_logical(t, 5)
        rr = lax.bitwise_and(t, 31)
        pos = lax.bitwise_or(
            lax.bitwise_or(lax.bitwise_and(o, 32),
                           lax.shift_left(lax.bitwise_and(o, 1), 4)),
            lax.shift_right_logical(lax.bitwise_and(o, 31), 1))
        rows = rr * 16 + iota16
        d = plsc.load_gather(acc_v, [rows, jnp.full((16,), pos, jnp.int32)])
        acct_v[o, pl.ds(rr * 16, 16)] = d

    pltpu.sync_copy(acct_v, out_hbm.at[:, pl.ds(base, BPW)])


@jax.jit
def _sc_call(x, fp_bf, borders_pad, invlen):
    mesh = plsc.VectorSubcoreMesh(core_axis_name="c", subcore_axis_name="s",
                                  num_cores=NC, num_subcores=NS)
    f = pl.kernel(
        _lookup_body,
        out_type=jax.ShapeDtypeStruct((OUT, B), jnp.float32),
        mesh=mesh,
        compiler_params=pltpu.CompilerParams(needs_layout_passes=False,
                                             use_tc_tiling_on_sc=False),
        scratch_types=[
            pltpu.VMEM((4112,), jnp.float32),
            pltpu.VMEM((G,), jnp.float32),
            pltpu.VMEM((2, BPW), jnp.float32),
            pltpu.VMEM((2, BPW), jnp.int32),
            pltpu.VMEM((2, BPW), jnp.int32),
            pltpu.VMEM((2, BPW), jnp.float32),
            pltpu.VMEM((BPW, OUT), jnp.bfloat16),
            pltpu.VMEM((BPW, OUT), jnp.bfloat16),
            pltpu.VMEM((BPW, OUT), jnp.float32),
            pltpu.VMEM((OUT, BPW), jnp.float32),
            pltpu.SemaphoreType.DMA,
            pltpu.SemaphoreType.DMA,
            pltpu.SemaphoreType.DMA,
        ],
    )
    return f(x, fp_bf, borders_pad, invlen)


def kernel(x, func_parameter, borders, inverse_chunk_lengths):
    fp_bf = (jnp.swapaxes(func_parameter, 1, 2)
             .reshape(ROWS * IN, OUT).astype(jnp.bfloat16))
    borders_pad = jnp.pad(borders, (0, 4112 - ROWS))
    return _sc_call(x, fp_bf, borders_pad, inverse_chunk_lengths)


# speedup vs baseline: 1.0031x; 1.0031x over previous
import jax
import jax.numpy as jnp
from jax import lax
from jax.experimental import pallas as pl
from jax.experimental.pallas import tpu as pltpu
from jax.experimental.pallas import tpu_sc as plsc

G = 4096
IN = 64
OUT = 64
B = 16384
NC = 2
NS = 16
NW = NC * NS
BPW = B // NW
BLK = 128
NBLK = BPW // BLK
ROWS = G + 1


def _lookup_body(x_hbm, fpbf_hbm, borders_hbm, invlen_hbm, out_hbm,
                 borders_v, invlen_v, x_v, idxl_v, idxr_v, delta_v,
                 bufl_v, bufr_v, acc_v, acct_v, semx, seml, semr):
    wid = lax.axis_index("s") * NC + lax.axis_index("c")
    base = wid * BPW
    iota16 = lax.iota(jnp.int32, 16)

    pltpu.sync_copy(borders_hbm, borders_v)
    pltpu.sync_copy(invlen_hbm, invlen_v)

    zeros16 = jnp.zeros((16,), jnp.float32)

    @plsc.parallel_loop(0, BPW, unroll=4)
    def _(b):
        for r in range(OUT // 16):
            acc_v[b, pl.ds(r * 16, 16)] = zeros16

    def compute_weights(i1, par):
        def wgt_body(j, cc):
            xv = x_v[par, pl.ds(j * 16, 16)]
            e = jnp.exp(-jnp.abs(xv))
            cdf = jnp.where(xv > 0.0, 1.0 - 0.5 * e, 0.5 * e)
            idx = jnp.clip((cdf * float(G)).astype(jnp.int32), 0, G - 1)
            left = plsc.load_gather(borders_v, [idx])
            invl = plsc.load_gather(invlen_v, [idx])
            delta_v[par, pl.ds(j * 16, 16)] = (xv - left) * invl
            row = idx * IN + i1
            idxl_v[par, pl.ds(j * 16, 16)] = row
            idxr_v[par, pl.ds(j * 16, 16)] = row + IN
            return cc
        lax.fori_loop(0, BPW // 16, wgt_body, 0)

    def fire_block(par, blk):
        pltpu.async_copy(
            fpbf_hbm.at[idxl_v.at[par, pl.ds(blk * BLK, BLK)]],
            bufl_v.at[pl.ds(blk * BLK, BLK)], seml)
        pltpu.async_copy(
            fpbf_hbm.at[idxr_v.at[par, pl.ds(blk * BLK, BLK)]],
            bufr_v.at[pl.ds(blk * BLK, BLK)], semr)

    def wait_block(par, blk):
        pltpu.make_async_copy(
            fpbf_hbm.at[idxl_v.at[par, pl.ds(blk * BLK, BLK)]],
            bufl_v.at[pl.ds(blk * BLK, BLK)], seml).wait()
        pltpu.make_async_copy(
            fpbf_hbm.at[idxr_v.at[par, pl.ds(blk * BLK, BLK)]],
            bufr_v.at[pl.ds(blk * BLK, BLK)], semr).wait()

    pltpu.sync_copy(x_hbm.at[0, pl.ds(base, BPW)], x_v.at[0])
    pltpu.async_copy(x_hbm.at[1, pl.ds(base, BPW)], x_v.at[1], semx)
    compute_weights(0, 0)
    for blk in range(NBLK):
        fire_block(0, blk)

    def dim_body(i, c):
        par = lax.rem(i, 2)
        parn = 1 - par

        @pl.when(i < IN - 1)
        def _():
            pltpu.make_async_copy(
                x_hbm.at[i + 1, pl.ds(base, BPW)], x_v.at[parn], semx).wait()

            @pl.when(i < IN - 2)
            def _():
                pltpu.async_copy(
                    x_hbm.at[i + 2, pl.ds(base, BPW)], x_v.at[par], semx)

            compute_weights(i + 1, parn)

        for blk in range(NBLK):
            wait_block(par, blk)

            @plsc.parallel_loop(0, BLK, unroll=4)
            def _(j):
                b = blk * BLK + j
                d = plsc.load_gather(
                    delta_v.at[par], [jnp.full((16,), b, jnp.int32)])
                for grp in range(2):
                    lw = plsc.bitcast(bufl_v[b, pl.ds(grp * 32, 32)],
                                      jnp.int32)
                    rw = plsc.bitcast(bufr_v[b, pl.ds(grp * 32, 32)],
                                      jnp.int32)
                    for half in range(2):
                        if half == 0:
                            li = lax.shift_left(lw, 16)
                            ri = lax.shift_left(rw, 16)
                        else:
                            li = lax.bitwise_and(lw, jnp.int32(-65536))
                            ri = lax.bitwise_and(rw, jnp.int32(-65536))
                        L = plsc.bitcast(li, jnp.float32)
                        R = plsc.bitcast(ri, jnp.float32)
                        r = grp * 2 + half
                        plsc.addupdate(acc_v.at[b, pl.ds(r * 16, 16)],
                                       L + d * (R - L))

            @pl.when(i < IN - 1)
            def _():
                fire_block(parn, blk)

        return c

    lax.fori_loop(0, IN, dim_body, 0)

    @plsc.parallel_loop(0, OUT * (BPW // 16), unroll=4)
    def _(t):
        o = lax.shift_right_logical(t, 5)
        rr = lax.bitwise_and(t, 31)
        pos = lax.bitwise_or(
            lax.bitwise_or(lax.bitwise_and(o, 32),
                           lax.shift_left(lax.bitwise_and(o, 1), 4)),
            lax.shift_right_logical(lax.bitwise_and(o, 31), 1))
        rows = rr * 16 + iota16
        d = plsc.load_gather(acc_v, [rows, jnp.full((16,), pos, jnp.int32)])
        acct_v[o, pl.ds(rr * 16, 16)] = d

    pltpu.sync_copy(acct_v, out_hbm.at[:, pl.ds(base, BPW)])


@jax.jit
def _sc_call(x, fp_bf, borders_pad, invlen):
    mesh = plsc.VectorSubcoreMesh(core_axis_name="c", subcore_axis_name="s",
                                  num_cores=NC, num_subcores=NS)
    f = pl.kernel(
        _lookup_body,
        out_type=jax.ShapeDtypeStruct((OUT, B), jnp.float32),
        mesh=mesh,
        compiler_params=pltpu.CompilerParams(needs_layout_passes=False,
                                             use_tc_tiling_on_sc=False),
        scratch_types=[
            pltpu.VMEM((4112,), jnp.float32),
            pltpu.VMEM((G,), jnp.float32),
            pltpu.VMEM((2, BPW), jnp.float32),
            pltpu.VMEM((2, BPW), jnp.int32),
            pltpu.VMEM((2, BPW), jnp.int32),
            pltpu.VMEM((2, BPW), jnp.float32),
            pltpu.VMEM((BPW, OUT), jnp.bfloat16),
            pltpu.VMEM((BPW, OUT), jnp.bfloat16),
            pltpu.VMEM((BPW, OUT), jnp.float32),
            pltpu.VMEM((OUT, BPW), jnp.float32),
            pltpu.SemaphoreType.DMA,
            pltpu.SemaphoreType.DMA,
            pltpu.SemaphoreType.DMA,
        ],
    )
    return f(x, fp_bf, borders_pad, invlen)


def kernel(x, func_parameter, borders, inverse_chunk_lengths):
    fp_bf = (jnp.swapaxes(func_parameter, 1, 2)
             .reshape(ROWS * IN, OUT).astype(jnp.bfloat16))
    borders_pad = jnp.pad(borders, (0, 4112 - ROWS))
    return _sc_call(x, fp_bf, borders_pad, inverse_chunk_lengths)
